# baseline (speedup 1.0000x reference)
import jax
import jax.numpy as jnp
from jax import lax
from jax.experimental import pallas as pl
from jax.experimental.pallas import tpu as pltpu
from jax.experimental.pallas import tpu_sc as plsc

N = 10000
D = 128
E = 320000
NEG = 0.2

NPAD = 10112
NW = 32
BLK = 64
EB = 162
CHUNK = EB * BLK
EPAD = CHUNK * NW
NDEN = 80
RT = NPAD + NDEN
RT = 10240
ROWS_T = RT // 16
MMB = 1264


def _mm2_body(x_ref, wl_ref, wr_ref, xl_ref, xr_ref):
    x = x_ref[...]
    xl_ref[...] = jnp.dot(x, wl_ref[...], preferred_element_type=jnp.float32)
    xr_ref[...] = jnp.dot(x, wr_ref[...], preferred_element_type=jnp.float32)


def _mm2(x, wl, wr):
    return pl.pallas_call(
        _mm2_body,
        grid=(NPAD // MMB,),
        in_specs=[pl.BlockSpec((MMB, D), lambda i: (i, 0)),
                  pl.BlockSpec((D, D), lambda i: (0, 0)),
                  pl.BlockSpec((D, D), lambda i: (0, 0))],
        out_specs=[pl.BlockSpec((MMB, D), lambda i: (i, 0)),
                   pl.BlockSpec((MMB, D), lambda i: (i, 0))],
        out_shape=[jax.ShapeDtypeStruct((NPAD, D), jnp.float32)] * 2,
    )(x, wl, wr)


def _norm_mm_body(ad_ref, den_ref, b_ref, wl_ref, wr_ref, xl_ref, xr_ref):
    a = ad_ref[0] + ad_ref[1]
    dvec = den_ref[0] + den_ref[1]
    h = jnp.maximum(a / (dvec + 1e-16) + b_ref[...], 0.0)
    xl_ref[...] = jnp.dot(h, wl_ref[...], preferred_element_type=jnp.float32)
    xr_ref[...] = jnp.dot(h, wr_ref[...], preferred_element_type=jnp.float32)


def _norm_mm(ad, den, b, wl, wr):
    return pl.pallas_call(
        _norm_mm_body,
        grid=(NPAD // MMB,),
        in_specs=[pl.BlockSpec((2, MMB, D), lambda i: (0, i, 0)),
                  pl.BlockSpec((2, MMB, 1), lambda i: (0, i, 0)),
                  pl.BlockSpec((1, D), lambda i: (0, 0)),
                  pl.BlockSpec((D, D), lambda i: (0, 0)),
                  pl.BlockSpec((D, D), lambda i: (0, 0))],
        out_specs=[pl.BlockSpec((MMB, D), lambda i: (i, 0)),
                   pl.BlockSpec((MMB, D), lambda i: (i, 0))],
        out_shape=[jax.ShapeDtypeStruct((NPAD, D), jnp.float32)] * 2,
    )(ad, den, b, wl, wr)


def _norm_body(ad_ref, den_ref, b_ref, o_ref):
    a = ad_ref[0] + ad_ref[1]
    dvec = den_ref[0] + den_ref[1]
    o_ref[...] = a / (dvec + 1e-16) + b_ref[...]


def _norm(ad, den, b):
    return pl.pallas_call(
        _norm_body,
        grid=(NPAD // MMB,),
        in_specs=[pl.BlockSpec((2, MMB, D), lambda i: (0, i, 0)),
                  pl.BlockSpec((2, MMB, 1), lambda i: (0, i, 0)),
                  pl.BlockSpec((1, D), lambda i: (0, 0))],
        out_specs=pl.BlockSpec((MMB, D), lambda i: (i, 0)),
        out_shape=jax.ShapeDtypeStruct((NPAD, D), jnp.float32),
    )(ad, den, b)


def _edge_body(xl_hbm, xr_hbm, src_hbm, dst_hbm, att_hbm, zad_hbm,
               ad_out,
               src_a, dst_a, xl_a, xr_a, src_b, dst_b, xl_b, xr_b,
               den_v, didx_v, att_v, tile_v, ad_sh, sem_a, sem_b, sem_sa, sem_sb):
    c = lax.axis_index("c")
    s = lax.axis_index("s")
    wid = s * 2 + c
    r0s = list(range(0, ROWS_T - BLK, BLK)) + [ROWS_T - BLK]

    pltpu.sync_copy(zad_hbm.at[pl.ds(0, BLK)], xl_a)
    pltpu.sync_copy(att_hbm, att_v)
    for r0 in r0s:
        pltpu.sync_copy(xl_a, ad_sh.at[pl.ds(s * ROWS_T + r0, BLK)])
    plsc.subcore_barrier()

    att_regs = [att_v[pl.ds(k * 16, 16)] for k in range(8)]
    lane = lax.iota(jnp.int32, 16)
    zeros16 = jnp.zeros((16,), jnp.int32)
    zerosf = jnp.zeros((16,), jnp.float32)
    lane0 = lane == 0

    def dz_body(i, c0):
        den_v[lax.div(i, jnp.int32(8)), pl.ds(lax.rem(i, jnp.int32(8)) * 16, 16)] = zerosf
        return c0

    lax.fori_loop(0, NDEN * 8, dz_body, 0)

    def di_body(i, c0):
        didx_v[pl.ds(i * 16, 16)] = jnp.broadcast_to(jnp.int32(NPAD), (16,)) + i * 16 + lane
        return c0

    lax.fori_loop(0, NDEN // 16, di_body, 0)

    bufs = ((src_a, dst_a, xl_a, xr_a, sem_a, sem_sa),
            (src_b, dst_b, xl_b, xr_b, sem_b, sem_sb))

    def start_gathers(bi, buf):
        src_v, dst_v, xl_rows, xr_rows, sem, sem_s = buf
        base = (wid * EB + bi) * BLK
        pltpu.sync_copy(src_hbm.at[pl.ds(base, BLK)], src_v)
        pltpu.sync_copy(dst_hbm.at[pl.ds(base, BLK)], dst_v)
        pltpu.make_async_copy(xl_hbm.at[src_v], xl_rows, sem).start()
        pltpu.make_async_copy(xr_hbm.at[dst_v], xr_rows, sem).start()

    def wait_gathers(buf):
        src_v, dst_v, xl_rows, xr_rows, sem, sem_s = buf
        pltpu.make_async_copy(xl_hbm.at[src_v], xl_rows, sem).wait()
        pltpu.make_async_copy(xr_hbm.at[dst_v], xr_rows, sem).wait()

    def compute_block(buf):
        src_v, dst_v, xl_rows, xr_rows, sem, sem_s = buf

        def group_body(g, c1):
            def edge_body(j, c2):
                e = g * 16 + j
                va = [jnp.zeros((16,), jnp.float32)] * 2
                for k in range(8):
                    aa = xl_rows[e, pl.ds(k * 16, 16)]
                    bb = xr_rows[e, pl.ds(k * 16, 16)]
                    t = aa + bb
                    lr = jnp.maximum(t, t * NEG)
                    va[k % 2] = va[k % 2] + lr * att_regs[k]
                plsc.store_scatter(tile_v, [lane, jnp.broadcast_to(j, (16,))],
                                   va[0] + va[1])
                return c2

            lax.fori_loop(0, 16, edge_body, 0)
            tsum = tile_v[0, :]
            for k in range(1, 16):
                tsum = tsum + tile_v[k, :]
            ex16 = jnp.exp(tsum)
            plsc.store_scatter(tile_v, [zeros16, lane], ex16)

            def scale_body(j, c3):
                e = g * 16 + j
                exv = plsc.load_gather(tile_v, [zeros16,
                                                jnp.broadcast_to(j, (16,))])
                for k in range(8):
                    xl_rows[e, pl.ds(k * 16, 16)] = (
                        xl_rows[e, pl.ds(k * 16, 16)] * exv)
                dstv = plsc.load_gather(dst_v, [jnp.broadcast_to(e, (16,))])
                rowv = lax.shift_right_logical(dstv, jnp.broadcast_to(jnp.int32(7), (16,)))
                colv = jnp.bitwise_and(dstv, jnp.broadcast_to(jnp.int32(127), (16,)))
                cur = plsc.load_gather(den_v, [rowv, colv])
                plsc.store_scatter(den_v, [rowv, colv], cur + exv, mask=lane0)
                return c3

            lax.fori_loop(0, 16, scale_body, 0)
            return c1

        lax.fori_loop(0, BLK // 16, group_body, 0)

    def start_scatter(buf):
        src_v, dst_v, xl_rows, xr_rows, sem, sem_s = buf
        pltpu.make_async_copy(xl_rows, ad_sh.at[dst_v], sem_s).start(add=True)

    def wait_scatter(buf):
        src_v, dst_v, xl_rows, xr_rows, sem, sem_s = buf
        pltpu.make_async_copy(xl_rows, ad_sh.at[dst_v], sem_s).wait()

    start_gathers(0, bufs[0])
    wait_gathers(bufs[0])
    start_gathers(1, bufs[1])
    compute_block(bufs[0])
    start_scatter(bufs[0])
    wait_gathers(bufs[1])
    wait_scatter(bufs[0])
    start_gathers(2, bufs[0])
    compute_block(bufs[1])
    start_scatter(bufs[1])

    def pair_body(p, carry):
        for b in range(2):
            bi = 2 + p * 2 + b
            cur, nxt = bufs[b], bufs[1 - b]
            wait_gathers(cur)
            wait_scatter(nxt)
            start_gathers(bi + 1, nxt)
            compute_block(cur)
            start_scatter(cur)
        return carry

    lax.fori_loop(0, (EB - 2) // 2, pair_body, 0)
    wait_gathers(bufs[0])
    wait_scatter(bufs[1])

    pltpu.sync_copy(den_v, ad_sh.at[didx_v], add=True)
    plsc.subcore_barrier()
    for r0 in r0s:
        pltpu.sync_copy(ad_sh.at[pl.ds(s * ROWS_T + r0, BLK)], xl_a)
        pltpu.sync_copy(xl_a, ad_out.at[c, pl.ds(s * ROWS_T + r0, BLK)])


_EDGE_CALL_CACHE = []


def _edge_call(*args):
    if not _EDGE_CALL_CACHE:
        _EDGE_CALL_CACHE.append(_make_edge_call())
    return _EDGE_CALL_CACHE[0](*args)


def _make_edge_call():
    return pl.kernel(
        _edge_body,
        out_type=[jax.ShapeDtypeStruct((2, RT, D), jnp.float32)],
        mesh=plsc.VectorSubcoreMesh(core_axis_name="c", subcore_axis_name="s"),
        compiler_params=pltpu.CompilerParams(needs_layout_passes=False),
        scratch_types=[
            pltpu.VMEM((BLK,), jnp.int32),
            pltpu.VMEM((BLK,), jnp.int32),
            pltpu.VMEM((BLK, D), jnp.float32),
            pltpu.VMEM((BLK, D), jnp.float32),
            pltpu.VMEM((BLK,), jnp.int32),
            pltpu.VMEM((BLK,), jnp.int32),
            pltpu.VMEM((BLK, D), jnp.float32),
            pltpu.VMEM((BLK, D), jnp.float32),
            pltpu.VMEM((NDEN, D), jnp.float32),
            pltpu.VMEM((NDEN,), jnp.int32),
            pltpu.VMEM((D,), jnp.float32),
            pltpu.VMEM((16, 16), jnp.float32),
            pltpu.VMEM_SHARED((RT, D), jnp.float32),
            pltpu.SemaphoreType.DMA,
            pltpu.SemaphoreType.DMA,
            pltpu.SemaphoreType.DMA,
            pltpu.SemaphoreType.DMA,
        ],
    )


def kernel(x, edge_index, Wl1, Wr1, att1, b1, Wl2, Wr2, att2, b2):
    xp = jnp.zeros((NPAD, D), jnp.float32).at[:N].set(x)
    loop = jnp.arange(N, dtype=jnp.int32)
    padi = jnp.full((EPAD + BLK - E - N,), N, jnp.int32)
    src = jnp.concatenate([edge_index[0], loop, padi])
    dst = jnp.concatenate([edge_index[1], loop, padi])
    zad = jnp.zeros((BLK, D), jnp.float32)
    b1r = b1.reshape(1, D)
    b2r = b2.reshape(1, D)

    xl1, xr1 = _mm2(xp, Wl1, Wr1)
    (ad1,) = _edge_call(xl1, xr1, src, dst, att1, zad)
    den1 = ad1[:, NPAD:NPAD + NPAD // D, :].reshape(2, NPAD, 1)
    xl2, xr2 = _norm_mm(ad1, den1, b1r, Wl2, Wr2)
    (ad2,) = _edge_call(xl2, xr2, src, dst, att2, zad)
    den2 = ad2[:, NPAD:NPAD + NPAD // D, :].reshape(2, NPAD, 1)
    out = _norm(ad2, den2, b2r)
    return out[:N]


# speedup vs baseline: 1.0152x; 1.0152x over previous
import jax
import jax.numpy as jnp
from jax import lax
from jax.experimental import pallas as pl
from jax.experimental.pallas import tpu as pltpu
from jax.experimental.pallas import tpu_sc as plsc

N = 10000
D = 128
E = 320000
NEG = 0.2

NPAD = 10112
NW = 32
BLK = 64
EB = 162
CHUNK = EB * BLK
EPAD = CHUNK * NW
NDEN = 80
RT = NPAD + NDEN
RT = 10240
ROWS_T = RT // 16
MMB = 1264


def _mm2_body(x_ref, wl_ref, wr_ref, xl_ref, xr_ref):
    x = x_ref[...]
    xl_ref[...] = jnp.dot(x, wl_ref[...], preferred_element_type=jnp.float32)
    xr_ref[...] = jnp.dot(x, wr_ref[...], preferred_element_type=jnp.float32)


def _mm2(x, wl, wr):
    return pl.pallas_call(
        _mm2_body,
        grid=(NPAD // MMB,),
        in_specs=[pl.BlockSpec((MMB, D), lambda i: (i, 0)),
                  pl.BlockSpec((D, D), lambda i: (0, 0)),
                  pl.BlockSpec((D, D), lambda i: (0, 0))],
        out_specs=[pl.BlockSpec((MMB, D), lambda i: (i, 0)),
                   pl.BlockSpec((MMB, D), lambda i: (i, 0))],
        out_shape=[jax.ShapeDtypeStruct((NPAD, D), jnp.float32)] * 2,
    )(x, wl, wr)


def _norm_mm_body(ad_ref, den_ref, b_ref, wl_ref, wr_ref, xl_ref, xr_ref):
    a = ad_ref[0] + ad_ref[1]
    dvec = den_ref[0] + den_ref[1]
    h = jnp.maximum(a / (dvec + 1e-16) + b_ref[...], 0.0)
    xl_ref[...] = jnp.dot(h, wl_ref[...], preferred_element_type=jnp.float32)
    xr_ref[...] = jnp.dot(h, wr_ref[...], preferred_element_type=jnp.float32)


def _norm_mm(ad, den, b, wl, wr):
    return pl.pallas_call(
        _norm_mm_body,
        grid=(NPAD // MMB,),
        in_specs=[pl.BlockSpec((2, MMB, D), lambda i: (0, i, 0)),
                  pl.BlockSpec((2, MMB, 1), lambda i: (0, i, 0)),
                  pl.BlockSpec((1, D), lambda i: (0, 0)),
                  pl.BlockSpec((D, D), lambda i: (0, 0)),
                  pl.BlockSpec((D, D), lambda i: (0, 0))],
        out_specs=[pl.BlockSpec((MMB, D), lambda i: (i, 0)),
                   pl.BlockSpec((MMB, D), lambda i: (i, 0))],
        out_shape=[jax.ShapeDtypeStruct((NPAD, D), jnp.float32)] * 2,
    )(ad, den, b, wl, wr)


def _norm_body(ad_ref, den_ref, b_ref, o_ref):
    a = ad_ref[0] + ad_ref[1]
    dvec = den_ref[0] + den_ref[1]
    o_ref[...] = a / (dvec + 1e-16) + b_ref[...]


def _norm(ad, den, b):
    return pl.pallas_call(
        _norm_body,
        grid=(NPAD // MMB,),
        in_specs=[pl.BlockSpec((2, MMB, D), lambda i: (0, i, 0)),
                  pl.BlockSpec((2, MMB, 1), lambda i: (0, i, 0)),
                  pl.BlockSpec((1, D), lambda i: (0, 0))],
        out_specs=pl.BlockSpec((MMB, D), lambda i: (i, 0)),
        out_shape=jax.ShapeDtypeStruct((NPAD, D), jnp.float32),
    )(ad, den, b)


def _edge_body(xl_hbm, xr_hbm, src_hbm, dst_hbm, att_hbm, zad_hbm,
               ad_out,
               src_a, dst_a, xl_a, xr_a, src_b, dst_b, xl_b, xr_b,
               den_v, didx_v, att_v, tile_v, ad_sh, sem_a, sem_b):
    c = lax.axis_index("c")
    s = lax.axis_index("s")
    wid = s * 2 + c
    r0s = list(range(0, ROWS_T - BLK, BLK)) + [ROWS_T - BLK]

    pltpu.sync_copy(zad_hbm.at[pl.ds(0, BLK)], xl_a)
    pltpu.sync_copy(att_hbm, att_v)
    for r0 in r0s:
        pltpu.sync_copy(xl_a, ad_sh.at[pl.ds(s * ROWS_T + r0, BLK)])
    plsc.subcore_barrier()

    att_regs = [att_v[pl.ds(k * 16, 16)] for k in range(8)]
    lane = lax.iota(jnp.int32, 16)
    zeros16 = jnp.zeros((16,), jnp.int32)
    zerosf = jnp.zeros((16,), jnp.float32)
    lane0 = lane == 0

    def dz_body(i, c0):
        den_v[lax.div(i, jnp.int32(8)), pl.ds(lax.rem(i, jnp.int32(8)) * 16, 16)] = zerosf
        return c0

    lax.fori_loop(0, NDEN * 8, dz_body, 0)

    def di_body(i, c0):
        didx_v[pl.ds(i * 16, 16)] = jnp.broadcast_to(jnp.int32(NPAD), (16,)) + i * 16 + lane
        return c0

    lax.fori_loop(0, NDEN // 16, di_body, 0)

    bufs = ((src_a, dst_a, xl_a, xr_a, sem_a),
            (src_b, dst_b, xl_b, xr_b, sem_b))

    def start_gathers(bi, buf):
        src_v, dst_v, xl_rows, xr_rows, sem = buf
        base = (wid * EB + bi) * BLK
        pltpu.sync_copy(src_hbm.at[pl.ds(base, BLK)], src_v)
        pltpu.sync_copy(dst_hbm.at[pl.ds(base, BLK)], dst_v)
        pltpu.make_async_copy(xl_hbm.at[src_v], xl_rows, sem).start()
        pltpu.make_async_copy(xr_hbm.at[dst_v], xr_rows, sem).start()

    def wait_gathers(buf):
        src_v, dst_v, xl_rows, xr_rows, sem = buf
        pltpu.make_async_copy(xl_hbm.at[src_v], xl_rows, sem).wait()
        pltpu.make_async_copy(xr_hbm.at[dst_v], xr_rows, sem).wait()

    def compute_block(buf):
        src_v, dst_v, xl_rows, xr_rows, sem = buf

        def group_body(g, c1):
            def edge_body(j, c2):
                e = g * 16 + j
                vacc = jnp.zeros((16,), jnp.float32)
                for k in range(8):
                    aa = xl_rows[e, pl.ds(k * 16, 16)]
                    bb = xr_rows[e, pl.ds(k * 16, 16)]
                    t = aa + bb
                    lr = jnp.maximum(t, t * NEG)
                    vacc = vacc + lr * att_regs[k]
                plsc.store_scatter(tile_v, [lane, jnp.broadcast_to(j, (16,))],
                                   vacc)
                return c2

            lax.fori_loop(0, 16, edge_body, 0)
            tsum = tile_v[0, :]
            for k in range(1, 16):
                tsum = tsum + tile_v[k, :]
            ex16 = jnp.exp(tsum)
            plsc.store_scatter(tile_v, [zeros16, lane], ex16)

            def scale_body(j, c3):
                e = g * 16 + j
                exv = plsc.load_gather(tile_v, [zeros16,
                                                jnp.broadcast_to(j, (16,))])
                for k in range(8):
                    xl_rows[e, pl.ds(k * 16, 16)] = (
                        xl_rows[e, pl.ds(k * 16, 16)] * exv)
                dstv = plsc.load_gather(dst_v, [jnp.broadcast_to(e, (16,))])
                rowv = lax.shift_right_logical(dstv, jnp.broadcast_to(jnp.int32(7), (16,)))
                colv = jnp.bitwise_and(dstv, jnp.broadcast_to(jnp.int32(127), (16,)))
                cur = plsc.load_gather(den_v, [rowv, colv])
                plsc.store_scatter(den_v, [rowv, colv], cur + exv, mask=lane0)
                return c3

            lax.fori_loop(0, 16, scale_body, 0)
            return c1

        lax.fori_loop(0, BLK // 16, group_body, 0)

    def scatter_block(buf):
        src_v, dst_v, xl_rows, xr_rows, sem = buf
        pltpu.sync_copy(xl_rows, ad_sh.at[dst_v], add=True)

    start_gathers(0, bufs[0])

    def pair_body(p, carry):
        for b in range(2):
            bi = p * 2 + b
            cur, nxt = bufs[b], bufs[1 - b]
            wait_gathers(cur)
            start_gathers(bi + 1, nxt)
            compute_block(cur)
            scatter_block(cur)
        return carry

    lax.fori_loop(0, EB // 2 - 1, pair_body, 0)
    for b in range(2):
        bi = EB - 2 + b
        cur, nxt = bufs[b], bufs[1 - b]
        wait_gathers(cur)
        if b == 0:
            start_gathers(bi + 1, nxt)
        compute_block(cur)
        scatter_block(cur)

    pltpu.sync_copy(den_v, ad_sh.at[didx_v], add=True)
    plsc.subcore_barrier()
    for r0 in r0s:
        pltpu.sync_copy(ad_sh.at[pl.ds(s * ROWS_T + r0, BLK)], xl_a)
        pltpu.sync_copy(xl_a, ad_out.at[c, pl.ds(s * ROWS_T + r0, BLK)])


_EDGE_CALL_CACHE = []


def _edge_call(*args):
    if not _EDGE_CALL_CACHE:
        _EDGE_CALL_CACHE.append(_make_edge_call())
    return _EDGE_CALL_CACHE[0](*args)


def _make_edge_call():
    return pl.kernel(
        _edge_body,
        out_type=[jax.ShapeDtypeStruct((2, RT, D), jnp.float32)],
        mesh=plsc.VectorSubcoreMesh(core_axis_name="c", subcore_axis_name="s"),
        compiler_params=pltpu.CompilerParams(needs_layout_passes=False),
        scratch_types=[
            pltpu.VMEM((BLK,), jnp.int32),
            pltpu.VMEM((BLK,), jnp.int32),
            pltpu.VMEM((BLK, D), jnp.float32),
            pltpu.VMEM((BLK, D), jnp.float32),
            pltpu.VMEM((BLK,), jnp.int32),
            pltpu.VMEM((BLK,), jnp.int32),
            pltpu.VMEM((BLK, D), jnp.float32),
            pltpu.VMEM((BLK, D), jnp.float32),
            pltpu.VMEM((NDEN, D), jnp.float32),
            pltpu.VMEM((NDEN,), jnp.int32),
            pltpu.VMEM((D,), jnp.float32),
            pltpu.VMEM((16, 16), jnp.float32),
            pltpu.VMEM_SHARED((RT, D), jnp.float32),
            pltpu.SemaphoreType.DMA,
            pltpu.SemaphoreType.DMA,
        ],
    )


def kernel(x, edge_index, Wl1, Wr1, att1, b1, Wl2, Wr2, att2, b2):
    xp = jnp.zeros((NPAD, D), jnp.float32).at[:N].set(x)
    loop = jnp.arange(N, dtype=jnp.int32)
    padi = jnp.full((EPAD + BLK - E - N,), N, jnp.int32)
    src = jnp.concatenate([edge_index[0], loop, padi])
    dst = jnp.concatenate([edge_index[1], loop, padi])
    zad = jnp.zeros((BLK, D), jnp.float32)
    b1r = b1.reshape(1, D)
    b2r = b2.reshape(1, D)

    xl1, xr1 = _mm2(xp, Wl1, Wr1)
    (ad1,) = _edge_call(xl1, xr1, src, dst, att1, zad)
    den1 = ad1[:, NPAD:NPAD + NPAD // D, :].reshape(2, NPAD, 1)
    xl2, xr2 = _norm_mm(ad1, den1, b1r, Wl2, Wr2)
    (ad2,) = _edge_call(xl2, xr2, src, dst, att2, zad)
    den2 = ad2[:, NPAD:NPAD + NPAD // D, :].reshape(2, NPAD, 1)
    out = _norm(ad2, den2, b2r)
    return out[:N]
